# baseline (speedup 1.0000x reference)
import functools

import jax
import jax.numpy as jnp
from jax import lax
from jax.experimental import pallas as pl
from jax.experimental.pallas import tpu as pltpu

_LANES = 128


def _iota2(shape, offset=0):
    r = lax.broadcasted_iota(jnp.int32, shape, 0)
    c = lax.broadcasted_iota(jnp.int32, shape, 1)
    return r * _LANES + c + offset


def _bit_mask(shape, t):
    if t < 7:
        c = lax.broadcasted_iota(jnp.int32, shape, 1)
        return (c & (1 << t)) != 0
    r = lax.broadcasted_iota(jnp.int32, shape, 0)
    return (r & (1 << (t - 7))) != 0


def _take_small(bit, desc):
    if desc is None:
        return ~bit
    if isinstance(desc, bool):
        return bit if desc else ~bit
    return bit == desc


def _partner(a, t, bit):
    d = 1 << t
    if t < 7:
        ax, n, sh = 1, _LANES, d
    else:
        ax, n, sh = 0, a.shape[0], d >> 7
    return jnp.where(bit, pltpu.roll(a, sh, ax), pltpu.roll(a, n - sh, ax))


def _ce(k, v, t, desc, tie):
    bit = _bit_mask(k.shape, t)
    ts = _take_small(bit, desc)
    pk = _partner(k, t, bit)
    if v is None:
        mn = jnp.minimum(k, pk)
        mx = jnp.maximum(k, pk)
        return jnp.where(ts, mn, mx), None
    pv = _partner(v, t, bit)
    if tie:
        less = (k < pk) | ((k == pk) & (v < pv))
    else:
        less = k < pk
    keep = less == ts
    return jnp.where(keep, k, pk), jnp.where(keep, v, pv)


def _local_stage(k, v, s, desc, tie, lgc):
    for t in range(min(s - 1, lgc - 1), -1, -1):
        k, v = _ce(k, v, t, desc, tie)
    return k, v


def _sort_refs(streams, pad_rows, chunk_rows, lgn, lgc, unroll=1):
    nch = pad_rows // chunk_rows
    shape = (chunk_rows, _LANES)

    def load(i, c_sl):
        k_ref, v_ref, _ = streams[i]
        return (k_ref[c_sl, :], v_ref[c_sl, :] if v_ref is not None else None)

    def store(i, c_sl, k, v):
        k_ref, v_ref, _ = streams[i]
        k_ref[c_sl, :] = k
        if v_ref is not None:
            v_ref[c_sl, :] = v

    def chunk_body(c, _):
        sl = pl.ds(c * chunk_rows, chunk_rows)
        data = [load(i, sl) for i in range(len(streams))]
        for s in range(1, lgc + 1):
            if s < lgc:
                desc = _bit_mask(shape, s)
            else:
                desc = (c & 1) == 1
            for t in range(min(s - 1, lgc - 1), -1, -1):
                data = [_ce(k, v, t, desc, streams[i][2])
                        for i, (k, v) in enumerate(data)]
        for i, (k, v) in enumerate(data):
            store(i, sl, k, v)
        return 0

    lax.fori_loop(0, nch, chunk_body, 0, unroll=unroll)

    for s in range(lgc + 1, lgn + 1):
        for t in range(s - 1, lgc - 1, -1):
            dch = 1 << (t - lgc)
            for c in range(nch):
                if c & dch:
                    continue
                p = c + dch
                desc = ((c >> (s - lgc)) & 1) == 1
                sa = slice(c * chunk_rows, (c + 1) * chunk_rows)
                sb = slice(p * chunk_rows, (p + 1) * chunk_rows)
                for i, (_, v_ref, tie) in enumerate(streams):
                    ka, va = load(i, sa)
                    kb, vb = load(i, sb)
                    if tie:
                        less = (ka < kb) | ((ka == kb) & (va < vb))
                    else:
                        less = ka < kb
                    if desc:
                        less = ~less
                    has_v = v_ref is not None
                    store(i, sa, jnp.where(less, ka, kb),
                          jnp.where(less, va, vb) if has_v else None)
                    store(i, sb, jnp.where(less, kb, ka),
                          jnp.where(less, vb, va) if has_v else None)

        def chunk_body2(c, _, s=s):
            sl = pl.ds(c * chunk_rows, chunk_rows)
            data = [load(i, sl) for i in range(len(streams))]
            desc = ((c >> (s - lgc)) & 1) == 1
            data = [_local_stage(k, v, s, desc, streams[i][2], lgc)
                    for i, (k, v) in enumerate(data)]
            for i, (k, v) in enumerate(data):
                store(i, sl, k, v)
            return 0

        lax.fori_loop(0, nch, chunk_body2, 0, unroll=unroll)


def _efdm_row_kernel(x_ref, y_ref, oi_ref, oy_ref, kx, ix, ky, *, rows,
                     pad_rows, chunk_rows, lgn, lgc):
    pad = pad_rows - rows
    n = rows * _LANES
    kx[0:rows, :] = x_ref[0]
    ix[...] = _iota2((pad_rows, _LANES))
    ky[0:rows, :] = y_ref[0]
    if pad:
        inf_pad = jnp.full((pad, _LANES), jnp.inf, jnp.float32)
        kx[rows:pad_rows, :] = inf_pad
        ky[rows:pad_rows, :] = inf_pad

    _sort_refs([(kx, ix, True), (ky, None, False)],
               pad_rows, chunk_rows, lgn, lgc, unroll=3)

    del n
    oi_ref[0] = ix[0:rows, :]
    oy_ref[0] = ky[0:rows, :]


def _efdm_sort_call(x3, y3, rows, pad_rows, chunk_rows, lgn, lgc,
                    interpret=False):
    nrows = x3.shape[0]
    body = functools.partial(_efdm_row_kernel, rows=rows, pad_rows=pad_rows,
                             chunk_rows=chunk_rows, lgn=lgn, lgc=lgc)
    return pl.pallas_call(
        body,
        grid=(nrows,),
        in_specs=[
            pl.BlockSpec((1, rows, _LANES), lambda i: (i, 0, 0)),
            pl.BlockSpec((1, rows, _LANES), lambda i: (i, 0, 0)),
        ],
        out_specs=[
            pl.BlockSpec((1, rows, _LANES), lambda i: (i, 0, 0)),
            pl.BlockSpec((1, rows, _LANES), lambda i: (i, 0, 0)),
        ],
        out_shape=[
            jax.ShapeDtypeStruct((nrows, rows, _LANES), jnp.int32),
            jax.ShapeDtypeStruct((nrows, rows, _LANES), jnp.float32),
        ],
        scratch_shapes=[
            pltpu.VMEM((pad_rows, _LANES), jnp.float32),
            pltpu.VMEM((pad_rows, _LANES), jnp.int32),
            pltpu.VMEM((pad_rows, _LANES), jnp.float32),
        ],
        compiler_params=pltpu.CompilerParams(
            dimension_semantics=("arbitrary",),
        ),
        interpret=interpret,
    )(x3, y3)


def _make_sc_scatter(n_rows_total, n_row):
    from jax.experimental.pallas import tpu_sc as plsc

    info = plsc.get_sparse_core_info()
    nc, ns = info.num_cores, info.num_subcores
    share = n_row // ns
    assert n_row % ns == 0 and share % 8 == 0
    rows_per_sc = n_rows_total // nc
    mesh = plsc.VectorSubcoreMesh(core_axis_name="c", subcore_axis_name="s")

    @functools.partial(
        pl.kernel,
        mesh=mesh,
        out_type=jax.ShapeDtypeStruct((n_rows_total * n_row,), jnp.float32),
        scratch_types=[
            pltpu.VMEM((share,), jnp.int32),
            pltpu.VMEM((share,), jnp.float32),
            pltpu.VMEM_SHARED((n_row,), jnp.float32),
            pltpu.SemaphoreType.DMA,
        ],
    )
    def scatter_kernel(idx_hbm, val_hbm, out_hbm, idx_v, val_v, rowbuf, sem):
        c = lax.axis_index("c")
        s = lax.axis_index("s")

        def row_body(i, _):
            r = i * nc + c
            off = r * n_row + s * share
            pltpu.sync_copy(idx_hbm.at[pl.ds(off, share)], idx_v)
            pltpu.sync_copy(val_hbm.at[pl.ds(off, share)], val_v)
            pltpu.async_copy(val_v, rowbuf.at[idx_v], sem).wait()
            plsc.subcore_barrier()
            pltpu.sync_copy(rowbuf.at[pl.ds(s * share, share)], val_v)
            pltpu.sync_copy(val_v, out_hbm.at[pl.ds(off, share)])
            plsc.subcore_barrier()
            return 0

        lax.fori_loop(0, rows_per_sc, row_body, 0)

    return scatter_kernel


def kernel(x, y):
    B, C, W, H = x.shape
    n = W * H
    rows = n // _LANES
    lgn = (n - 1).bit_length()
    pad_rows = (1 << lgn) // _LANES
    chunk_rows = 64
    lgc = (chunk_rows * _LANES).bit_length() - 1

    x3 = x.reshape(B * C, rows, _LANES)
    y3 = y.reshape(B * C, rows, _LANES)
    gidx, ysort = _efdm_sort_call(x3, y3, rows, pad_rows, chunk_rows, lgn, lgc)
    n_total = B * C * n
    scatter = _make_sc_scatter(B * C, n)
    out = scatter(gidx.reshape(n_total), ysort.reshape(n_total))
    return out.reshape(B, C, W, H)


# speedup vs baseline: 1.0696x; 1.0696x over previous
import functools

import jax
import jax.numpy as jnp
from jax import lax
from jax.experimental import pallas as pl
from jax.experimental.pallas import tpu as pltpu

_LANES = 128


def _iota2(shape, offset=0):
    r = lax.broadcasted_iota(jnp.int32, shape, 0)
    c = lax.broadcasted_iota(jnp.int32, shape, 1)
    return r * _LANES + c + offset


def _bit_mask(shape, t):
    if t < 7:
        c = lax.broadcasted_iota(jnp.int32, shape, 1)
        return (c & (1 << t)) != 0
    r = lax.broadcasted_iota(jnp.int32, shape, 0)
    return (r & (1 << (t - 7))) != 0


def _take_small(bit, desc):
    if desc is None:
        return ~bit
    if isinstance(desc, bool):
        return bit if desc else ~bit
    return bit == desc


def _partner(a, t, bit):
    d = 1 << t
    if t < 7:
        ax, n, sh = 1, _LANES, d
    else:
        ax, n, sh = 0, a.shape[0], d >> 7
    return jnp.where(bit, pltpu.roll(a, sh, ax), pltpu.roll(a, n - sh, ax))


def _ce(k, v, t, desc, tie):
    bit = _bit_mask(k.shape, t)
    ts = _take_small(bit, desc)
    pk = _partner(k, t, bit)
    if v is None:
        mn = jnp.minimum(k, pk)
        mx = jnp.maximum(k, pk)
        return jnp.where(ts, mn, mx), None
    pv = _partner(v, t, bit)
    if tie:
        less = (k < pk) | ((k == pk) & (v < pv))
    else:
        less = k < pk
    keep = less == ts
    return jnp.where(keep, k, pk), jnp.where(keep, v, pv)


def _local_stage(k, v, s, desc, tie, lgc):
    for t in range(min(s - 1, lgc - 1), -1, -1):
        k, v = _ce(k, v, t, desc, tie)
    return k, v


def _sort_refs(streams, pad_rows, chunk_rows, lgn, lgc, unroll=1,
               active=None):
    nch = pad_rows // chunk_rows
    if active is None:
        active = nch
    shape = (chunk_rows, _LANES)

    def load(i, c_sl):
        k_ref, v_ref, _ = streams[i]
        return (k_ref[c_sl, :], v_ref[c_sl, :] if v_ref is not None else None)

    def store(i, c_sl, k, v):
        k_ref, v_ref, _ = streams[i]
        k_ref[c_sl, :] = k
        if v_ref is not None:
            v_ref[c_sl, :] = v

    def chunk_body(c, _):
        sl = pl.ds(c * chunk_rows, chunk_rows)
        data = [load(i, sl) for i in range(len(streams))]
        for s in range(1, lgc + 1):
            if s < lgc:
                desc = _bit_mask(shape, s)
            else:
                desc = (c & 1) == 1
            for t in range(min(s - 1, lgc - 1), -1, -1):
                data = [_ce(k, v, t, desc, streams[i][2])
                        for i, (k, v) in enumerate(data)]
        for i, (k, v) in enumerate(data):
            store(i, sl, k, v)
        return 0

    lax.fori_loop(0, active, chunk_body, 0, unroll=unroll)

    for s in range(lgc + 1, lgn + 1):
        for t in range(s - 1, lgc - 1, -1):
            dch = 1 << (t - lgc)
            for c in range(nch):
                if c & dch:
                    continue
                p = c + dch
                if p >= active:
                    continue
                desc = ((c >> (s - lgc)) & 1) == 1
                sa = slice(c * chunk_rows, (c + 1) * chunk_rows)
                sb = slice(p * chunk_rows, (p + 1) * chunk_rows)
                for i, (_, v_ref, tie) in enumerate(streams):
                    ka, va = load(i, sa)
                    kb, vb = load(i, sb)
                    if tie:
                        less = (ka < kb) | ((ka == kb) & (va < vb))
                    else:
                        less = ka < kb
                    if desc:
                        less = ~less
                    has_v = v_ref is not None
                    store(i, sa, jnp.where(less, ka, kb),
                          jnp.where(less, va, vb) if has_v else None)
                    store(i, sb, jnp.where(less, kb, ka),
                          jnp.where(less, vb, va) if has_v else None)

        def chunk_body2(c, _, s=s):
            sl = pl.ds(c * chunk_rows, chunk_rows)
            data = [load(i, sl) for i in range(len(streams))]
            desc = ((c >> (s - lgc)) & 1) == 1
            data = [_local_stage(k, v, s, desc, streams[i][2], lgc)
                    for i, (k, v) in enumerate(data)]
            for i, (k, v) in enumerate(data):
                store(i, sl, k, v)
            return 0

        lax.fori_loop(0, active, chunk_body2, 0, unroll=unroll)


def _efdm_row_kernel(x_ref, y_ref, oi_ref, oy_ref, kx, ix, ky, *, rows,
                     pad_rows, chunk_rows, lgn, lgc):
    pad = pad_rows - rows
    n = rows * _LANES
    kx[0:rows, :] = x_ref[0]
    ix[...] = _iota2((pad_rows, _LANES))
    ky[0:rows, :] = y_ref[0]
    if pad:
        inf_pad = jnp.full((pad, _LANES), jnp.inf, jnp.float32)
        kx[rows:pad_rows, :] = inf_pad
        ky[rows:pad_rows, :] = inf_pad

    _sort_refs([(kx, ix, True), (ky, None, False)],
               pad_rows, chunk_rows, lgn, lgc, unroll=2)

    del n
    oi_ref[0] = ix[0:rows, :]
    oy_ref[0] = ky[0:rows, :]


def _efdm_sort_call(x3, y3, rows, pad_rows, chunk_rows, lgn, lgc,
                    interpret=False):
    nrows = x3.shape[0]
    body = functools.partial(_efdm_row_kernel, rows=rows, pad_rows=pad_rows,
                             chunk_rows=chunk_rows, lgn=lgn, lgc=lgc)
    return pl.pallas_call(
        body,
        grid=(nrows,),
        in_specs=[
            pl.BlockSpec((1, rows, _LANES), lambda i: (i, 0, 0)),
            pl.BlockSpec((1, rows, _LANES), lambda i: (i, 0, 0)),
        ],
        out_specs=[
            pl.BlockSpec((1, rows, _LANES), lambda i: (i, 0, 0)),
            pl.BlockSpec((1, rows, _LANES), lambda i: (i, 0, 0)),
        ],
        out_shape=[
            jax.ShapeDtypeStruct((nrows, rows, _LANES), jnp.int32),
            jax.ShapeDtypeStruct((nrows, rows, _LANES), jnp.float32),
        ],
        scratch_shapes=[
            pltpu.VMEM((pad_rows, _LANES), jnp.float32),
            pltpu.VMEM((pad_rows, _LANES), jnp.int32),
            pltpu.VMEM((pad_rows, _LANES), jnp.float32),
        ],
        compiler_params=pltpu.CompilerParams(
            dimension_semantics=("arbitrary",),
        ),
        interpret=interpret,
    )(x3, y3)


def _make_sc_scatter(n_rows_total, n_row):
    from jax.experimental.pallas import tpu_sc as plsc

    info = plsc.get_sparse_core_info()
    nc, ns = info.num_cores, info.num_subcores
    share = n_row // ns
    assert n_row % ns == 0 and share % 8 == 0
    rows_per_sc = n_rows_total // nc
    mesh = plsc.VectorSubcoreMesh(core_axis_name="c", subcore_axis_name="s")

    @functools.partial(
        pl.kernel,
        mesh=mesh,
        out_type=jax.ShapeDtypeStruct((n_rows_total * n_row,), jnp.float32),
        scratch_types=[
            pltpu.VMEM((share,), jnp.int32),
            pltpu.VMEM((share,), jnp.float32),
            pltpu.VMEM_SHARED((n_row,), jnp.float32),
            pltpu.SemaphoreType.DMA,
        ],
    )
    def scatter_kernel(idx_hbm, val_hbm, out_hbm, idx_v, val_v, rowbuf, sem):
        c = lax.axis_index("c")
        s = lax.axis_index("s")

        def row_body(i, _):
            r = i * nc + c
            off = r * n_row + s * share
            pltpu.sync_copy(idx_hbm.at[pl.ds(off, share)], idx_v)
            pltpu.sync_copy(val_hbm.at[pl.ds(off, share)], val_v)
            pltpu.async_copy(val_v, rowbuf.at[idx_v], sem).wait()
            plsc.subcore_barrier()
            pltpu.sync_copy(rowbuf.at[pl.ds(s * share, share)], val_v)
            pltpu.sync_copy(val_v, out_hbm.at[pl.ds(off, share)])
            plsc.subcore_barrier()
            return 0

        lax.fori_loop(0, rows_per_sc, row_body, 0)

    return scatter_kernel


def kernel(x, y):
    B, C, W, H = x.shape
    n = W * H
    rows = n // _LANES
    lgn = (n - 1).bit_length()
    pad_rows = (1 << lgn) // _LANES
    chunk_rows = 64
    lgc = (chunk_rows * _LANES).bit_length() - 1

    x3 = x.reshape(B * C, rows, _LANES)
    y3 = y.reshape(B * C, rows, _LANES)
    gidx, ysort = _efdm_sort_call(x3, y3, rows, pad_rows, chunk_rows, lgn, lgc)
    n_total = B * C * n
    scatter = _make_sc_scatter(B * C, n)
    out = scatter(gidx.reshape(n_total), ysort.reshape(n_total))
    return out.reshape(B, C, W, H)


# speedup vs baseline: 1.1437x; 1.0693x over previous
import functools

import jax
import jax.numpy as jnp
from jax import lax
from jax.experimental import pallas as pl
from jax.experimental.pallas import tpu as pltpu

_LANES = 128


def _iota2(shape, offset=0):
    r = lax.broadcasted_iota(jnp.int32, shape, 0)
    c = lax.broadcasted_iota(jnp.int32, shape, 1)
    return r * _LANES + c + offset


def _bit_mask(shape, t):
    if t < 7:
        c = lax.broadcasted_iota(jnp.int32, shape, 1)
        return (c & (1 << t)) != 0
    r = lax.broadcasted_iota(jnp.int32, shape, 0)
    return (r & (1 << (t - 7))) != 0


def _take_small(bit, desc):
    if desc is None:
        return ~bit
    if isinstance(desc, bool):
        return bit if desc else ~bit
    return bit == desc


def _partner(a, t, bit):
    d = 1 << t
    if t < 7:
        ax, n, sh = 1, _LANES, d
    else:
        ax, n, sh = 0, a.shape[0], d >> 7
    return jnp.where(bit, pltpu.roll(a, sh, ax), pltpu.roll(a, n - sh, ax))


def _ce(k, v, t, desc, tie):
    bit = _bit_mask(k.shape, t)
    ts = _take_small(bit, desc)
    pk = _partner(k, t, bit)
    if v is None:
        mn = jnp.minimum(k, pk)
        mx = jnp.maximum(k, pk)
        return jnp.where(ts, mn, mx), None
    pv = _partner(v, t, bit)
    if tie:
        less = (k < pk) | ((k == pk) & (v < pv))
    else:
        less = k < pk
    keep = less == ts
    return jnp.where(keep, k, pk), jnp.where(keep, v, pv)


def _local_stage(k, v, s, desc, tie, lgc):
    for t in range(min(s - 1, lgc - 1), -1, -1):
        k, v = _ce(k, v, t, desc, tie)
    return k, v


def _sort_refs(streams, pad_rows, chunk_rows, lgn, lgc, unroll=1,
               active=None):
    nch = pad_rows // chunk_rows
    if active is None:
        active = nch
    shape = (chunk_rows, _LANES)

    def load(i, c_sl):
        k_ref, v_ref, _ = streams[i]
        return (k_ref[c_sl, :], v_ref[c_sl, :] if v_ref is not None else None)

    def store(i, c_sl, k, v):
        k_ref, v_ref, _ = streams[i]
        k_ref[c_sl, :] = k
        if v_ref is not None:
            v_ref[c_sl, :] = v

    def chunk_body(c, _):
        sl = pl.ds(c * chunk_rows, chunk_rows)
        data = [load(i, sl) for i in range(len(streams))]
        for s in range(1, lgc + 1):
            if s < lgc:
                desc = _bit_mask(shape, s)
            else:
                desc = (c & 1) == 1
            for t in range(min(s - 1, lgc - 1), -1, -1):
                data = [_ce(k, v, t, desc, streams[i][2])
                        for i, (k, v) in enumerate(data)]
        for i, (k, v) in enumerate(data):
            store(i, sl, k, v)
        return 0

    lax.fori_loop(0, active, chunk_body, 0, unroll=unroll)

    for s in range(lgc + 1, lgn + 1):
        for t in range(s - 1, lgc - 1, -1):
            dch = 1 << (t - lgc)
            for c in range(nch):
                if c & dch:
                    continue
                p = c + dch
                if p >= active:
                    continue
                desc = ((c >> (s - lgc)) & 1) == 1
                sa = slice(c * chunk_rows, (c + 1) * chunk_rows)
                sb = slice(p * chunk_rows, (p + 1) * chunk_rows)
                for i, (_, v_ref, tie) in enumerate(streams):
                    ka, va = load(i, sa)
                    kb, vb = load(i, sb)
                    if tie:
                        less = (ka < kb) | ((ka == kb) & (va < vb))
                    else:
                        less = ka < kb
                    if desc:
                        less = ~less
                    has_v = v_ref is not None
                    store(i, sa, jnp.where(less, ka, kb),
                          jnp.where(less, va, vb) if has_v else None)
                    store(i, sb, jnp.where(less, kb, ka),
                          jnp.where(less, vb, va) if has_v else None)

        def chunk_body2(c, _, s=s):
            sl = pl.ds(c * chunk_rows, chunk_rows)
            data = [load(i, sl) for i in range(len(streams))]
            desc = ((c >> (s - lgc)) & 1) == 1
            data = [_local_stage(k, v, s, desc, streams[i][2], lgc)
                    for i, (k, v) in enumerate(data)]
            for i, (k, v) in enumerate(data):
                store(i, sl, k, v)
            return 0

        lax.fori_loop(0, active, chunk_body2, 0, unroll=unroll)


def _efdm_row_kernel(x_ref, y_ref, oi_ref, oy_ref, kx, ix, ky, *, rows,
                     pad_rows, chunk_rows, lgn, lgc):
    pad = pad_rows - rows
    n = rows * _LANES
    kx[0:rows, :] = x_ref[0]
    ix[...] = _iota2((pad_rows, _LANES))
    ky[0:rows, :] = y_ref[0]
    if pad:
        inf_pad = jnp.full((pad, _LANES), jnp.inf, jnp.float32)
        kx[rows:pad_rows, :] = inf_pad
        ky[rows:pad_rows, :] = inf_pad

    _sort_refs([(kx, ix, True), (ky, None, False)],
               pad_rows, chunk_rows, lgn, lgc, unroll=1)

    del n
    oi_ref[0] = ix[0:rows, :]
    oy_ref[0] = ky[0:rows, :]


def _efdm_sort_call(x3, y3, rows, pad_rows, chunk_rows, lgn, lgc,
                    interpret=False):
    nrows = x3.shape[0]
    body = functools.partial(_efdm_row_kernel, rows=rows, pad_rows=pad_rows,
                             chunk_rows=chunk_rows, lgn=lgn, lgc=lgc)
    return pl.pallas_call(
        body,
        grid=(nrows,),
        in_specs=[
            pl.BlockSpec((1, rows, _LANES), lambda i: (i, 0, 0)),
            pl.BlockSpec((1, rows, _LANES), lambda i: (i, 0, 0)),
        ],
        out_specs=[
            pl.BlockSpec((1, rows, _LANES), lambda i: (i, 0, 0)),
            pl.BlockSpec((1, rows, _LANES), lambda i: (i, 0, 0)),
        ],
        out_shape=[
            jax.ShapeDtypeStruct((nrows, rows, _LANES), jnp.int32),
            jax.ShapeDtypeStruct((nrows, rows, _LANES), jnp.float32),
        ],
        scratch_shapes=[
            pltpu.VMEM((pad_rows, _LANES), jnp.float32),
            pltpu.VMEM((pad_rows, _LANES), jnp.int32),
            pltpu.VMEM((pad_rows, _LANES), jnp.float32),
        ],
        compiler_params=pltpu.CompilerParams(
            dimension_semantics=("arbitrary",),
        ),
        interpret=interpret,
    )(x3, y3)


def _make_sc_scatter(n_rows_total, n_row):
    from jax.experimental.pallas import tpu_sc as plsc

    info = plsc.get_sparse_core_info()
    nc, ns = info.num_cores, info.num_subcores
    share = n_row // ns
    assert n_row % ns == 0 and share % 8 == 0
    rows_per_sc = n_rows_total // nc
    mesh = plsc.VectorSubcoreMesh(core_axis_name="c", subcore_axis_name="s")

    @functools.partial(
        pl.kernel,
        mesh=mesh,
        out_type=jax.ShapeDtypeStruct((n_rows_total * n_row,), jnp.float32),
        scratch_types=[
            pltpu.VMEM((share,), jnp.int32),
            pltpu.VMEM((share,), jnp.float32),
            pltpu.VMEM_SHARED((n_row,), jnp.float32),
            pltpu.SemaphoreType.DMA,
        ],
    )
    def scatter_kernel(idx_hbm, val_hbm, out_hbm, idx_v, val_v, rowbuf, sem):
        c = lax.axis_index("c")
        s = lax.axis_index("s")

        def row_body(i, _):
            r = i * nc + c
            off = r * n_row + s * share
            pltpu.sync_copy(idx_hbm.at[pl.ds(off, share)], idx_v)
            pltpu.sync_copy(val_hbm.at[pl.ds(off, share)], val_v)
            pltpu.async_copy(val_v, rowbuf.at[idx_v], sem).wait()
            plsc.subcore_barrier()
            pltpu.sync_copy(rowbuf.at[pl.ds(s * share, share)], val_v)
            pltpu.sync_copy(val_v, out_hbm.at[pl.ds(off, share)])
            plsc.subcore_barrier()
            return 0

        lax.fori_loop(0, rows_per_sc, row_body, 0)

    return scatter_kernel


def kernel(x, y):
    B, C, W, H = x.shape
    n = W * H
    rows = n // _LANES
    lgn = (n - 1).bit_length()
    pad_rows = (1 << lgn) // _LANES
    chunk_rows = 128
    lgc = (chunk_rows * _LANES).bit_length() - 1

    x3 = x.reshape(B * C, rows, _LANES)
    y3 = y.reshape(B * C, rows, _LANES)
    gidx, ysort = _efdm_sort_call(x3, y3, rows, pad_rows, chunk_rows, lgn, lgc)
    n_total = B * C * n
    scatter = _make_sc_scatter(B * C, n)
    out = scatter(gidx.reshape(n_total), ysort.reshape(n_total))
    return out.reshape(B, C, W, H)


# speedup vs baseline: 1.1574x; 1.0121x over previous
import functools

import jax
import jax.numpy as jnp
from jax import lax
from jax.experimental import pallas as pl
from jax.experimental.pallas import tpu as pltpu

_LANES = 128


def _iota2(shape, offset=0):
    r = lax.broadcasted_iota(jnp.int32, shape, 0)
    c = lax.broadcasted_iota(jnp.int32, shape, 1)
    return r * _LANES + c + offset


def _bit_mask(shape, t):
    if t < 7:
        c = lax.broadcasted_iota(jnp.int32, shape, 1)
        return (c & (1 << t)) != 0
    r = lax.broadcasted_iota(jnp.int32, shape, 0)
    return (r & (1 << (t - 7))) != 0


def _take_small(bit, desc):
    if desc is None:
        return ~bit
    if isinstance(desc, bool):
        return bit if desc else ~bit
    return bit == desc


def _partner(a, t, bit):
    d = 1 << t
    if t < 7:
        ax, n, sh = 1, _LANES, d
    else:
        ax, n, sh = 0, a.shape[0], d >> 7
    return jnp.where(bit, pltpu.roll(a, sh, ax), pltpu.roll(a, n - sh, ax))


def _ce(k, v, t, desc, tie):
    bit = _bit_mask(k.shape, t)
    ts = _take_small(bit, desc)
    pk = _partner(k, t, bit)
    if v is None:
        mn = jnp.minimum(k, pk)
        mx = jnp.maximum(k, pk)
        return jnp.where(ts, mn, mx), None
    pv = _partner(v, t, bit)
    if tie:
        less = (k < pk) | ((k == pk) & (v < pv))
    else:
        less = k < pk
    keep = less == ts
    return jnp.where(keep, k, pk), jnp.where(keep, v, pv)


def _local_stage(k, v, s, desc, tie, lgc):
    for t in range(min(s - 1, lgc - 1), -1, -1):
        k, v = _ce(k, v, t, desc, tie)
    return k, v


def _sort_refs(streams, pad_rows, chunk_rows, lgn, lgc, unroll=1,
               active=None):
    nch = pad_rows // chunk_rows
    if active is None:
        active = nch
    shape = (chunk_rows, _LANES)

    def load(i, c_sl):
        k_ref, v_ref, _ = streams[i]
        return (k_ref[c_sl, :], v_ref[c_sl, :] if v_ref is not None else None)

    def store(i, c_sl, k, v):
        k_ref, v_ref, _ = streams[i]
        k_ref[c_sl, :] = k
        if v_ref is not None:
            v_ref[c_sl, :] = v

    def chunk_body(c, _):
        sl = pl.ds(c * chunk_rows, chunk_rows)
        data = [load(i, sl) for i in range(len(streams))]
        for s in range(1, lgc + 1):
            if s < lgc:
                desc = _bit_mask(shape, s)
            else:
                desc = (c & 1) == 1
            for t in range(min(s - 1, lgc - 1), -1, -1):
                data = [_ce(k, v, t, desc, streams[i][2])
                        for i, (k, v) in enumerate(data)]
        for i, (k, v) in enumerate(data):
            store(i, sl, k, v)
        return 0

    lax.fori_loop(0, active, chunk_body, 0, unroll=unroll)

    for s in range(lgc + 1, lgn + 1):
        for t in range(s - 1, lgc - 1, -1):
            dch = 1 << (t - lgc)
            for c in range(nch):
                if c & dch:
                    continue
                p = c + dch
                if p >= active:
                    continue
                desc = ((c >> (s - lgc)) & 1) == 1
                sa = slice(c * chunk_rows, (c + 1) * chunk_rows)
                sb = slice(p * chunk_rows, (p + 1) * chunk_rows)
                for i, (_, v_ref, tie) in enumerate(streams):
                    ka, va = load(i, sa)
                    kb, vb = load(i, sb)
                    if tie:
                        less = (ka < kb) | ((ka == kb) & (va < vb))
                    else:
                        less = ka < kb
                    if desc:
                        less = ~less
                    has_v = v_ref is not None
                    store(i, sa, jnp.where(less, ka, kb),
                          jnp.where(less, va, vb) if has_v else None)
                    store(i, sb, jnp.where(less, kb, ka),
                          jnp.where(less, vb, va) if has_v else None)

        def chunk_body2(c, _, s=s):
            sl = pl.ds(c * chunk_rows, chunk_rows)
            data = [load(i, sl) for i in range(len(streams))]
            desc = ((c >> (s - lgc)) & 1) == 1
            data = [_local_stage(k, v, s, desc, streams[i][2], lgc)
                    for i, (k, v) in enumerate(data)]
            for i, (k, v) in enumerate(data):
                store(i, sl, k, v)
            return 0

        lax.fori_loop(0, active, chunk_body2, 0, unroll=unroll)


def _efdm_row_kernel(x_ref, y_ref, oi_ref, oy_ref, kx, ix, ky, *, rows,
                     pad_rows, chunk_rows, lgn, lgc):
    pad = pad_rows - rows
    n = rows * _LANES
    kx[0:rows, :] = x_ref[0]
    ix[...] = _iota2((pad_rows, _LANES))
    ky[0:rows, :] = y_ref[0]
    if pad:
        inf_pad = jnp.full((pad, _LANES), jnp.inf, jnp.float32)
        kx[rows:pad_rows, :] = inf_pad
        ky[rows:pad_rows, :] = inf_pad

    _sort_refs([(kx, ix, True), (ky, None, False)],
               pad_rows, chunk_rows, lgn, lgc, unroll=2)

    del n
    oi_ref[0] = ix[0:rows, :]
    oy_ref[0] = ky[0:rows, :]


def _efdm_sort_call(x3, y3, rows, pad_rows, chunk_rows, lgn, lgc,
                    interpret=False):
    nrows = x3.shape[0]
    body = functools.partial(_efdm_row_kernel, rows=rows, pad_rows=pad_rows,
                             chunk_rows=chunk_rows, lgn=lgn, lgc=lgc)
    return pl.pallas_call(
        body,
        grid=(nrows,),
        in_specs=[
            pl.BlockSpec((1, rows, _LANES), lambda i: (i, 0, 0)),
            pl.BlockSpec((1, rows, _LANES), lambda i: (i, 0, 0)),
        ],
        out_specs=[
            pl.BlockSpec((1, rows, _LANES), lambda i: (i, 0, 0)),
            pl.BlockSpec((1, rows, _LANES), lambda i: (i, 0, 0)),
        ],
        out_shape=[
            jax.ShapeDtypeStruct((nrows, rows, _LANES), jnp.int32),
            jax.ShapeDtypeStruct((nrows, rows, _LANES), jnp.float32),
        ],
        scratch_shapes=[
            pltpu.VMEM((pad_rows, _LANES), jnp.float32),
            pltpu.VMEM((pad_rows, _LANES), jnp.int32),
            pltpu.VMEM((pad_rows, _LANES), jnp.float32),
        ],
        compiler_params=pltpu.CompilerParams(
            dimension_semantics=("arbitrary",),
        ),
        interpret=interpret,
    )(x3, y3)


def _make_sc_scatter(n_rows_total, n_row):
    from jax.experimental.pallas import tpu_sc as plsc

    info = plsc.get_sparse_core_info()
    nc, ns = info.num_cores, info.num_subcores
    share = n_row // ns
    assert n_row % ns == 0 and share % 8 == 0
    rows_per_sc = n_rows_total // nc
    mesh = plsc.VectorSubcoreMesh(core_axis_name="c", subcore_axis_name="s")

    @functools.partial(
        pl.kernel,
        mesh=mesh,
        out_type=jax.ShapeDtypeStruct((n_rows_total * n_row,), jnp.float32),
        scratch_types=[
            pltpu.VMEM((share,), jnp.int32),
            pltpu.VMEM((share,), jnp.float32),
            pltpu.VMEM_SHARED((n_row,), jnp.float32),
            pltpu.SemaphoreType.DMA,
        ],
    )
    def scatter_kernel(idx_hbm, val_hbm, out_hbm, idx_v, val_v, rowbuf, sem):
        c = lax.axis_index("c")
        s = lax.axis_index("s")

        def row_body(i, _):
            r = i * nc + c
            off = r * n_row + s * share
            pltpu.sync_copy(idx_hbm.at[pl.ds(off, share)], idx_v)
            pltpu.sync_copy(val_hbm.at[pl.ds(off, share)], val_v)
            pltpu.async_copy(val_v, rowbuf.at[idx_v], sem).wait()
            plsc.subcore_barrier()
            pltpu.sync_copy(rowbuf.at[pl.ds(s * share, share)], val_v)
            pltpu.sync_copy(val_v, out_hbm.at[pl.ds(off, share)])
            plsc.subcore_barrier()
            return 0

        lax.fori_loop(0, rows_per_sc, row_body, 0)

    return scatter_kernel


def kernel(x, y):
    B, C, W, H = x.shape
    n = W * H
    rows = n // _LANES
    lgn = (n - 1).bit_length()
    pad_rows = (1 << lgn) // _LANES
    chunk_rows = 128
    lgc = (chunk_rows * _LANES).bit_length() - 1

    x3 = x.reshape(B * C, rows, _LANES)
    y3 = y.reshape(B * C, rows, _LANES)
    gidx, ysort = _efdm_sort_call(x3, y3, rows, pad_rows, chunk_rows, lgn, lgc)
    n_total = B * C * n
    scatter = _make_sc_scatter(B * C, n)
    out = scatter(gidx.reshape(n_total), ysort.reshape(n_total))
    return out.reshape(B, C, W, H)
